# baseline (speedup 1.0000x reference)
import functools

import jax
import jax.numpy as jnp
from jax import lax
from jax.experimental import pallas as pl
from jax.experimental.pallas import tpu as pltpu
from jax.experimental.pallas import tpu_sc as plsc

D = 32
DPAD = 128
NI = 4096
NJ = 200
NC, NS = 2, 16
NW = NC * NS
SLAB = NI // NW
NBUF = 4
L = 16

_mesh = plsc.VectorSubcoreMesh(core_axis_name="c", subcore_axis_name="s")


@functools.partial(
    pl.kernel,
    out_type=jax.ShapeDtypeStruct((NJ, D, NI), jnp.float32),
    mesh=_mesh,
    scratch_types=(
        [pltpu.VMEM((NJ, SLAB), jnp.int32)]
        + [pltpu.VMEM((SLAB, DPAD), jnp.float32) for _ in range(NBUF)]
        + [pltpu.VMEM((D, SLAB), jnp.float32) for _ in range(NBUF)]
        + [pltpu.SemaphoreType.DMA for _ in range(2 * NBUF)]
    ),
    compiler_params=pltpu.CompilerParams(needs_layout_passes=False),
)
def _embed_gather(ids_hbm, table_hbm, out_hbm, ids_v, *rest):
    rows = rest[:NBUF]
    outs = rest[NBUF:2 * NBUF]
    gsems = rest[2 * NBUF:3 * NBUF]
    ssems = rest[3 * NBUF:]
    wid = lax.axis_index("s") * NC + lax.axis_index("c")
    base_i = wid * SLAB

    pltpu.sync_copy(ids_hbm.at[:, pl.ds(base_i, SLAB)], ids_v)

    def fire_gather(j, b):
        pltpu.async_copy(table_hbm.at[ids_v.at[j]], rows[b], gsems[b])

    def wait_gather(b):
        pltpu.make_async_copy(table_hbm.at[pl.ds(0, SLAB)], rows[b], gsems[b]).wait()

    def fire_store(j, b):
        pltpu.async_copy(outs[b], out_hbm.at[j, :, pl.ds(base_i, SLAB)], ssems[b])

    def wait_store(b):
        pltpu.make_async_copy(outs[b], out_hbm.at[0, :, pl.ds(0, SLAB)], ssems[b]).wait()

    ivecs = [lax.iota(jnp.int32, L) + m * L for m in range(SLAB // L)]
    kvecs = [jnp.full((L,), k, jnp.int32) for k in range(D)]

    def extract(b):
        for k in range(D):
            for m in range(SLAB // L):
                x = plsc.load_gather(rows[b], [ivecs[m], kvecs[k]])
                outs[b][k, pl.ds(m * L, L)] = x

    for b in range(NBUF):
        fire_gather(b, b)
    for b in range(NBUF):
        wait_gather(b)
        extract(b)
        fire_store(b, b)
        fire_gather(b + NBUF, b)

    @pl.loop(NBUF, NJ - NBUF, step=NBUF)
    def _pipeline(j0):
        for b in range(NBUF):
            j = j0 + b
            wait_gather(b)
            wait_store(b)
            extract(b)
            fire_store(j, b)
            fire_gather(j + NBUF, b)

    for b in range(NBUF):
        wait_gather(b)
        wait_store(b)
        extract(b)
        fire_store(NJ - NBUF + b, b)
    for b in range(NBUF):
        wait_store(b)


def kernel(input_ids, table):
    ids_t = input_ids.T
    tbl128 = jnp.pad(table, ((0, 0), (0, DPAD - D)))
    out_t = _embed_gather(ids_t, tbl128)
    return out_t.transpose(2, 0, 1)


# speedup vs baseline: 1.2395x; 1.2395x over previous
import functools

import jax
import jax.numpy as jnp
from jax import lax
from jax.experimental import pallas as pl
from jax.experimental.pallas import tpu as pltpu
from jax.experimental.pallas import tpu_sc as plsc

D = 32
DPAD = 128
NI = 4096
NJ = 200
NC, NS = 2, 16
NW = NC * NS
SLAB = NI // NW
NBUF = 4
L = 16

_mesh = plsc.VectorSubcoreMesh(core_axis_name="c", subcore_axis_name="s")


@functools.partial(
    pl.kernel,
    out_type=jax.ShapeDtypeStruct((NJ, D, NI), jnp.float32),
    mesh=_mesh,
    scratch_types=(
        [pltpu.VMEM((NJ, SLAB), jnp.int32)]
        + [pltpu.VMEM((SLAB, DPAD), jnp.float32) for _ in range(NBUF)]
        + [pltpu.VMEM((D, SLAB + 1), jnp.float32) for _ in range(NBUF)]
        + [pltpu.SemaphoreType.DMA for _ in range(2 * NBUF)]
    ),
    compiler_params=pltpu.CompilerParams(needs_layout_passes=False),
)
def _embed_gather(ids_hbm, table_hbm, out_hbm, ids_v, *rest):
    rows = rest[:NBUF]
    outs = rest[NBUF:2 * NBUF]
    gsems = rest[2 * NBUF:3 * NBUF]
    ssems = rest[3 * NBUF:]
    wid = lax.axis_index("s") * NC + lax.axis_index("c")
    base_i = wid * SLAB

    pltpu.sync_copy(ids_hbm.at[:, pl.ds(base_i, SLAB)], ids_v)

    def fire_gather(j, b):
        pltpu.async_copy(table_hbm.at[ids_v.at[j]], rows[b], gsems[b])

    def wait_gather(b):
        pltpu.make_async_copy(table_hbm.at[pl.ds(0, SLAB)], rows[b], gsems[b]).wait()

    def fire_store(j, b):
        pltpu.async_copy(
            outs[b].at[:, pl.ds(0, SLAB)],
            out_hbm.at[j, :, pl.ds(base_i, SLAB)],
            ssems[b],
        )

    def wait_store(b):
        pltpu.make_async_copy(
            outs[b].at[:, pl.ds(0, SLAB)],
            out_hbm.at[0, :, pl.ds(0, SLAB)],
            ssems[b],
        ).wait()

    kvecs = [lax.iota(jnp.int32, L) + kc * L for kc in range(D // L)]

    def extract(b):
        @pl.loop(0, SLAB, unroll=8)
        def _per_i(i):
            ivec = jnp.zeros((L,), jnp.int32) + i
            for kc in range(D // L):
                x = rows[b][i, pl.ds(kc * L, L)]
                plsc.store_scatter(outs[b], [kvecs[kc], ivec], x)

    for b in range(NBUF):
        fire_gather(b, b)
    for b in range(NBUF):
        wait_gather(b)
        extract(b)
        fire_store(b, b)
        fire_gather(b + NBUF, b)

    @pl.loop(NBUF, NJ - NBUF, step=NBUF)
    def _pipeline(j0):
        for b in range(NBUF):
            j = j0 + b
            wait_gather(b)
            wait_store(b)
            extract(b)
            fire_store(j, b)
            fire_gather(j + NBUF, b)

    for b in range(NBUF):
        wait_gather(b)
        wait_store(b)
        extract(b)
        fire_store(NJ - NBUF + b, b)
    for b in range(NBUF):
        wait_store(b)


def kernel(input_ids, table):
    ids_t = input_ids.T
    tbl128 = jnp.pad(table, ((0, 0), (0, DPAD - D)))
    out_t = _embed_gather(ids_t, tbl128)
    return out_t.transpose(2, 0, 1)


# speedup vs baseline: 1.2445x; 1.0040x over previous
import functools

import jax
import jax.numpy as jnp
from jax import lax
from jax.experimental import pallas as pl
from jax.experimental.pallas import tpu as pltpu
from jax.experimental.pallas import tpu_sc as plsc

D = 32
B = 4096 * 200
NC, NS = 2, 16
NW = NC * NS
BPW = B // NW
GSIZE = 128
CHUNK = 640
NG = CHUNK // GSIZE
NCHUNK = BPW // CHUNK
NBUF = 4
IDXROWS = BPW // GSIZE

_mesh = plsc.VectorSubcoreMesh(core_axis_name="c", subcore_axis_name="s")


@functools.partial(
    pl.kernel,
    out_type=jax.ShapeDtypeStruct((B, D), jnp.float32),
    mesh=_mesh,
    scratch_types=(
        [pltpu.VMEM((IDXROWS, GSIZE), jnp.int32)]
        + [pltpu.VMEM((CHUNK, D), jnp.float32) for _ in range(NBUF)]
        + [pltpu.SemaphoreType.DMA for _ in range(2 * NBUF)]
    ),
    compiler_params=pltpu.CompilerParams(use_tc_tiling_on_sc=False),
)
def _embed_gather(idx_hbm, table_hbm, out_hbm, idx_v, *rest):
    bufs = rest[:NBUF]
    gsems = rest[NBUF:2 * NBUF]
    ssems = rest[2 * NBUF:]
    wid = lax.axis_index("s") * NC + lax.axis_index("c")
    base = wid * BPW
    pltpu.sync_copy(idx_hbm.at[pl.ds(wid * IDXROWS, IDXROWS)], idx_v)

    def fire_gathers(g, b):
        for j in range(NG):
            pltpu.async_copy(
                table_hbm.at[idx_v.at[g * NG + j]],
                bufs[b].at[pl.ds(j * GSIZE, GSIZE)],
                gsems[b],
            )

    def wait_gathers(b):
        pltpu.make_async_copy(table_hbm.at[pl.ds(0, CHUNK)], bufs[b], gsems[b]).wait()

    def fire_store(g, b):
        pltpu.async_copy(bufs[b], out_hbm.at[pl.ds(base + g * CHUNK, CHUNK)], ssems[b])

    def wait_store(b):
        pltpu.make_async_copy(bufs[b], out_hbm.at[pl.ds(0, CHUNK)], ssems[b]).wait()

    for b in range(NBUF):
        fire_gathers(b, b)

    @pl.loop(0, NCHUNK - NBUF, step=NBUF)
    def _pipeline(g0):
        for b in range(NBUF):
            wait_gathers(b)
            fire_store(g0 + b, b)
        for b in range(NBUF):
            wait_store(b)
            fire_gathers(g0 + b + NBUF, b)

    for b in range(NBUF):
        wait_gathers(b)
        fire_store(NCHUNK - NBUF + b, b)
    for b in range(NBUF):
        wait_store(b)


def kernel(input_ids, table):
    idx = input_ids.reshape(NW * IDXROWS, GSIZE)
    out = _embed_gather(idx, table)
    return out.reshape(input_ids.shape[0], input_ids.shape[1], D)


# speedup vs baseline: 1.7272x; 1.3879x over previous
import functools

import jax
import jax.numpy as jnp
from jax import lax
from jax.experimental import pallas as pl
from jax.experimental.pallas import tpu as pltpu
from jax.experimental.pallas import tpu_sc as plsc

D = 32
NI = 4096
NJ = 200
NC, NS = 2, 16
NW = NC * NS
SLAB = NI // NW
NBUF = 8
L = 16
NM = SLAB // L
PITCH = SLAB + 1

_mesh = plsc.VectorSubcoreMesh(core_axis_name="c", subcore_axis_name="s")


@functools.partial(
    pl.kernel,
    out_type=jax.ShapeDtypeStruct((NJ, D // 8, D, 8, SLAB), jnp.float32),
    mesh=_mesh,
    scratch_types=(
        [pltpu.VMEM((SLAB, NJ), jnp.int32)]
        + [pltpu.VMEM((SLAB, D), jnp.float32) for _ in range(NBUF)]
        + [pltpu.VMEM((SLAB,), jnp.int32) for _ in range(NBUF)]
        + [pltpu.VMEM((D // 8, 8, PITCH), jnp.float32) for _ in range(NBUF)]
        + [pltpu.SemaphoreType.DMA for _ in range(2 * NBUF)]
    ),
    compiler_params=pltpu.CompilerParams(
        use_tc_tiling_on_sc=False, needs_layout_passes=False
    ),
)
def _embed_gather(ids_hbm, table_hbm, out_hbm, ids_v, *rest):
    rows = rest[:NBUF]
    jidx = rest[NBUF:2 * NBUF]
    outs = rest[2 * NBUF:3 * NBUF]
    gsems = rest[3 * NBUF:4 * NBUF]
    ssems = rest[4 * NBUF:]
    wid = lax.axis_index("s") * NC + lax.axis_index("c")

    pltpu.sync_copy(ids_hbm.at[pl.ds(wid * SLAB, SLAB)], ids_v)

    ivecs = [lax.iota(jnp.int32, L) + m * L for m in range(NM)]
    kbvecs = [lax.shift_right_logical(lax.iota(jnp.int32, L) + kc * L, 3)
              for kc in range(D // L)]
    rvecs = [lax.bitwise_and(lax.iota(jnp.int32, L) + kc * L,
                             jnp.full((L,), 7, jnp.int32))
             for kc in range(D // L)]

    def fire_gather(j, b):
        jvec = jnp.zeros((L,), jnp.int32) + j
        for m in range(NM):
            jidx[b][pl.ds(m * L, L)] = plsc.load_gather(ids_v, [ivecs[m], jvec])
        pltpu.async_copy(table_hbm.at[jidx[b]], rows[b], gsems[b])

    def wait_gather(b):
        pltpu.make_async_copy(table_hbm.at[pl.ds(0, SLAB)], rows[b], gsems[b]).wait()

    def fire_store(j, b):
        pltpu.async_copy(
            outs[b].at[:, :, pl.ds(0, SLAB)],
            out_hbm.at[j, :, wid],
            ssems[b],
        )

    def wait_store(b):
        pltpu.make_async_copy(
            outs[b].at[:, :, pl.ds(0, SLAB)],
            out_hbm.at[0, :, 0],
            ssems[b],
        ).wait()

    def extract(b):
        @pl.loop(0, SLAB, unroll=8)
        def _per_c(c):
            cvec = jnp.zeros((L,), jnp.int32) + c
            for kc in range(D // L):
                x = rows[b][c, pl.ds(kc * L, L)]
                plsc.store_scatter(outs[b], [kbvecs[kc], rvecs[kc], cvec], x)

    for b in range(NBUF):
        fire_gather(b, b)
    for b in range(NBUF):
        wait_gather(b)
        extract(b)
        fire_store(b, b)
        fire_gather(b + NBUF, b)

    @pl.loop(NBUF, NJ - NBUF, step=NBUF)
    def _pipeline(j0):
        for b in range(NBUF):
            j = j0 + b
            wait_gather(b)
            wait_store(b)
            extract(b)
            fire_store(j, b)
            fire_gather(j + NBUF, b)

    for b in range(NBUF):
        wait_gather(b)
        wait_store(b)
        extract(b)
        fire_store(NJ - NBUF + b, b)
    for b in range(NBUF):
        wait_store(b)


def kernel(input_ids, table):
    out5 = _embed_gather(input_ids, table)
    return out5.transpose(2, 4, 0, 1, 3).reshape(NI, NJ, D)
